# baseline (speedup 1.0000x reference)
import functools

import jax
import jax.numpy as jnp
from jax import lax
from jax.experimental import pallas as pl
from jax.experimental.pallas import tpu as pltpu
from jax.experimental.pallas import tpu_sc as plsc

VOCAB = 100000
HIDDEN = 128
EPS = 1e-6
LANES = 16
NBLK = HIDDEN // LANES

NC = 2
NS = 16
NW = NC * NS

CHUNK = 128
NBUF = 6
DIST = 5


def _rsqrt(m):
    i = lax.bitcast_convert_type(m, jnp.int32)
    i = 0x5F3759DF - lax.shift_right_logical(i, 1)
    y = lax.bitcast_convert_type(i, jnp.float32)
    h = m * 0.5
    for _ in range(2):
        y = y * (1.5 - h * y * y)
    return y


def _lane_sum_splat(x, scr, perms):
    x = x + jnp.flip(x)
    for p in perms:
        scr[...] = x
        x = x + plsc.load_gather(scr, [p])
    return x


def _make_kernel(n_rows, apply_w):
    assert n_rows % (NW * CHUNK) == 0
    rows_per_w = n_rows // NW
    nchunk = rows_per_w // CHUNK
    mesh = plsc.VectorSubcoreMesh(core_axis_name="c", subcore_axis_name="s")

    @functools.partial(
        pl.kernel,
        out_type=jax.ShapeDtypeStruct((n_rows, HIDDEN), jnp.float32),
        mesh=mesh,
        compiler_params=pltpu.CompilerParams(needs_layout_passes=False),
        scratch_types=[
            pltpu.VMEM((nchunk, CHUNK), jnp.int32),
            pltpu.VMEM((HIDDEN,), jnp.float32),
            pltpu.VMEM((NBUF, CHUNK, HIDDEN), jnp.float32),
            pltpu.VMEM((CHUNK, LANES), jnp.float32),
            pltpu.SemaphoreType.DMA((NBUF,)),
            pltpu.SemaphoreType.DMA((NBUF,)),
        ],
    )
    def k(table_hbm, idx_hbm, w_hbm, out_hbm, idx_v, w_v, bufs, scr, gsem, wsem):
        wid = lax.axis_index("s") * NC + lax.axis_index("c")
        base = wid * rows_per_w
        pltpu.sync_copy(idx_hbm.at[wid], idx_v)
        pltpu.sync_copy(w_hbm, w_v)

        lanes = lax.iota(jnp.int32, LANES)
        perms = [jnp.bitwise_xor(lanes, k) for k in (4, 2, 1)]

        def fire_gather(c):
            b = lax.rem(c, NBUF)
            half = CHUNK // 2
            for h in range(2):
                pltpu.async_copy(
                    table_hbm.at[idx_v.at[c, pl.ds(h * half, half)]],
                    bufs.at[b, pl.ds(h * half, half)],
                    gsem.at[b],
                )

        def wait_write(b):
            pltpu.make_async_copy(
                bufs.at[b], out_hbm.at[pl.ds(0, CHUNK)], wsem.at[b]
            ).wait()

        for c0 in range(DIST):
            fire_gather(c0)

        def step(c, carry):
            b = lax.rem(c, NBUF)
            buf = bufs.at[b]
            pltpu.make_async_copy(table_hbm.at[idx_v.at[c]], buf,
                                  gsem.at[b]).wait()

            @plsc.parallel_loop(0, CHUNK, unroll=1)
            def row_body(r):
                xs = []
                acc = None
                for j in range(NBLK):
                    x = buf[r, pl.ds(j * LANES, LANES)]
                    xs.append(x)
                    acc = x * x if acc is None else acc + x * x
                s = _lane_sum_splat(acc, scr.at[r], perms)
                y = _rsqrt(s * (1.0 / HIDDEN) + EPS)
                for j in range(NBLK):
                    if apply_w:
                        w = w_v[pl.ds(j * LANES, LANES)]
                        buf[r, pl.ds(j * LANES, LANES)] = xs[j] * (y * w)
                    else:
                        buf[r, pl.ds(j * LANES, LANES)] = xs[j] * y

            pltpu.async_copy(buf, out_hbm.at[pl.ds(base + c * CHUNK, CHUNK)],
                             wsem.at[b])

            @pl.when(c >= NBUF - DIST)
            def _():
                wait_write(lax.rem(c + DIST, NBUF))

            @pl.when(c + DIST < nchunk)
            def _():
                fire_gather(c + DIST)

            return carry

        lax.fori_loop(0, nchunk, step, 0)

        for c in range(nchunk - (NBUF - DIST), nchunk):
            wait_write(c % NBUF)

    return k


def kernel(input_ids, table, norm_weight):
    b, s = input_ids.shape
    n_rows = b * s
    idx3 = input_ids.astype(jnp.int32).reshape(NW, n_rows // (NW * CHUNK), CHUNK)
    out = lax.cond(
        jnp.all(norm_weight == 1.0),
        lambda: _make_kernel(n_rows, apply_w=False)(table, idx3, norm_weight),
        lambda: _make_kernel(n_rows, apply_w=True)(table, idx3, norm_weight),
    )
    return out.reshape(b, s, HIDDEN)


# speedup vs baseline: 1.0531x; 1.0531x over previous
import functools

import jax
import jax.numpy as jnp
from jax import lax
from jax.experimental import pallas as pl
from jax.experimental.pallas import tpu as pltpu
from jax.experimental.pallas import tpu_sc as plsc

VOCAB = 100000
HIDDEN = 128
EPS = 1e-6
LANES = 16
NBLK = HIDDEN // LANES

NC = 2
NS = 16
NW = NC * NS

CHUNK = 128
NBUF = 6
DIST = 5


def _rsqrt(m):
    i = lax.bitcast_convert_type(m, jnp.int32)
    i = 0x5F3759DF - lax.shift_right_logical(i, 1)
    y = lax.bitcast_convert_type(i, jnp.float32)
    h = m * 0.5
    for _ in range(2):
        y = y * (1.5 - h * y * y)
    return y


def _lane_sum_splat(x, scr, perms):
    x = x + jnp.flip(x)
    for p in perms:
        scr[...] = x
        x = x + plsc.load_gather(scr, [p])
    return x


def _make_kernel(n_rows, apply_w):
    assert n_rows % (NW * CHUNK) == 0
    rows_per_w = n_rows // NW
    nchunk = rows_per_w // CHUNK
    mesh = plsc.VectorSubcoreMesh(core_axis_name="c", subcore_axis_name="s")

    @functools.partial(
        pl.kernel,
        out_type=jax.ShapeDtypeStruct((n_rows, HIDDEN), jnp.float32),
        mesh=mesh,
        compiler_params=pltpu.CompilerParams(needs_layout_passes=False),
        scratch_types=[
            pltpu.VMEM((nchunk, CHUNK), jnp.int32),
            pltpu.VMEM((HIDDEN,), jnp.float32),
            pltpu.VMEM((NBUF, CHUNK, HIDDEN), jnp.float32),
            pltpu.VMEM((CHUNK, LANES), jnp.float32),
            pltpu.SemaphoreType.DMA((NBUF,)),
            pltpu.SemaphoreType.DMA((NBUF,)),
        ],
    )
    def k(table_hbm, idx_hbm, w_hbm, out_hbm, idx_v, w_v, bufs, scr, gsem, wsem):
        wid = lax.axis_index("s") * NC + lax.axis_index("c")
        base = wid * rows_per_w
        pltpu.sync_copy(idx_hbm.at[wid], idx_v)
        pltpu.sync_copy(w_hbm, w_v)

        lanes = lax.iota(jnp.int32, LANES)
        perms = [jnp.bitwise_xor(lanes, k) for k in (4, 2, 1)]

        def fire_gather(c):
            b = lax.rem(c, NBUF)
            pltpu.async_copy(table_hbm.at[idx_v.at[c]], bufs.at[b], gsem.at[b])

        def wait_write(b):
            pltpu.make_async_copy(
                bufs.at[b], out_hbm.at[pl.ds(0, CHUNK)], wsem.at[b]
            ).wait()

        for c0 in range(DIST):
            fire_gather(c0)

        def step(c, carry):
            b = lax.rem(c, NBUF)
            buf = bufs.at[b]
            pltpu.make_async_copy(table_hbm.at[idx_v.at[c]], buf,
                                  gsem.at[b]).wait()

            @plsc.parallel_loop(0, CHUNK, unroll=1)
            def row_body(r):
                xs = []
                acc = None
                for j in range(NBLK):
                    x = buf[r, pl.ds(j * LANES, LANES)]
                    xs.append(x)
                    acc = x * x if acc is None else acc + x * x
                s = jnp.sum(acc)
                y = _rsqrt(s * (1.0 / HIDDEN) + EPS)
                y = jnp.full((LANES,), y)
                for j in range(NBLK):
                    if apply_w:
                        w = w_v[pl.ds(j * LANES, LANES)]
                        buf[r, pl.ds(j * LANES, LANES)] = xs[j] * (y * w)
                    else:
                        buf[r, pl.ds(j * LANES, LANES)] = xs[j] * y

            pltpu.async_copy(buf, out_hbm.at[pl.ds(base + c * CHUNK, CHUNK)],
                             wsem.at[b])

            @pl.when(c >= NBUF - DIST)
            def _():
                wait_write(lax.rem(c + DIST, NBUF))

            @pl.when(c + DIST < nchunk)
            def _():
                fire_gather(c + DIST)

            return carry

        lax.fori_loop(0, nchunk, step, 0)

        for c in range(nchunk - (NBUF - DIST), nchunk):
            wait_write(c % NBUF)

    return k


def kernel(input_ids, table, norm_weight):
    b, s = input_ids.shape
    n_rows = b * s
    idx3 = input_ids.astype(jnp.int32).reshape(NW, n_rows // (NW * CHUNK), CHUNK)
    out = lax.cond(
        jnp.all(norm_weight == 1.0),
        lambda: _make_kernel(n_rows, apply_w=False)(table, idx3, norm_weight),
        lambda: _make_kernel(n_rows, apply_w=True)(table, idx3, norm_weight),
    )
    return out.reshape(b, s, HIDDEN)


# speedup vs baseline: 1.0538x; 1.0007x over previous
import functools

import jax
import jax.numpy as jnp
from jax import lax
from jax.experimental import pallas as pl
from jax.experimental.pallas import tpu as pltpu
from jax.experimental.pallas import tpu_sc as plsc

VOCAB = 100000
HIDDEN = 128
EPS = 1e-6
LANES = 16
NBLK = HIDDEN // LANES

NC = 2
NS = 16
NW = NC * NS

CHUNK = 128
NBUF = 6
DIST = 5


def _rsqrt(m):
    i = lax.bitcast_convert_type(m, jnp.int32)
    i = 0x5F3759DF - lax.shift_right_logical(i, 1)
    y = lax.bitcast_convert_type(i, jnp.float32)
    h = m * 0.5
    for _ in range(2):
        y = y * (1.5 - h * y * y)
    return y


def _make_kernel(n_rows, apply_w):
    assert n_rows % (NW * CHUNK) == 0
    rows_per_w = n_rows // NW
    nchunk = rows_per_w // CHUNK
    mesh = plsc.VectorSubcoreMesh(core_axis_name="c", subcore_axis_name="s")

    @functools.partial(
        pl.kernel,
        out_type=jax.ShapeDtypeStruct((n_rows, HIDDEN), jnp.float32),
        mesh=mesh,
        compiler_params=pltpu.CompilerParams(needs_layout_passes=False),
        scratch_types=[
            pltpu.VMEM((nchunk, CHUNK), jnp.int32),
            pltpu.VMEM((HIDDEN,), jnp.float32),
            pltpu.VMEM((NBUF, CHUNK, HIDDEN), jnp.float32),
            pltpu.SemaphoreType.DMA((NBUF,)),
            pltpu.SemaphoreType.DMA((NBUF,)),
        ],
    )
    def k(table_hbm, idx_hbm, w_hbm, out_hbm, idx_v, w_v, bufs, gsem, wsem):
        wid = lax.axis_index("s") * NC + lax.axis_index("c")
        base = wid * rows_per_w
        pltpu.sync_copy(idx_hbm.at[wid], idx_v)
        pltpu.sync_copy(w_hbm, w_v)

        def fire_gather(c):
            b = lax.rem(c, NBUF)
            pltpu.async_copy(table_hbm.at[idx_v.at[c]], bufs.at[b], gsem.at[b])

        def wait_write(b):
            pltpu.make_async_copy(
                bufs.at[b], out_hbm.at[pl.ds(0, CHUNK)], wsem.at[b]
            ).wait()

        for c0 in range(DIST):
            fire_gather(c0)

        def step(c, carry):
            b = lax.rem(c, NBUF)
            buf = bufs.at[b]
            pltpu.make_async_copy(table_hbm.at[idx_v.at[c]], buf,
                                  gsem.at[b]).wait()

            @plsc.parallel_loop(0, CHUNK, unroll=1)
            def row_body(r):
                xs = []
                acc = None
                for j in range(NBLK):
                    x = buf[r, pl.ds(j * LANES, LANES)]
                    xs.append(x)
                    acc = x * x if acc is None else acc + x * x
                s = jnp.sum(acc)
                y = _rsqrt(s * (1.0 / HIDDEN) + EPS)
                y = jnp.full((LANES,), y)
                for j in range(NBLK):
                    if apply_w:
                        w = w_v[pl.ds(j * LANES, LANES)]
                        buf[r, pl.ds(j * LANES, LANES)] = xs[j] * (y * w)
                    else:
                        buf[r, pl.ds(j * LANES, LANES)] = xs[j] * y

            pltpu.async_copy(buf, out_hbm.at[pl.ds(base + c * CHUNK, CHUNK)],
                             wsem.at[b])

            @pl.when(c >= NBUF - DIST)
            def _():
                wait_write(lax.rem(c + DIST, NBUF))

            @pl.when(c + DIST < nchunk)
            def _():
                fire_gather(c + DIST)

            return carry

        lax.fori_loop(0, nchunk, step, 0)

        for c in range(nchunk - (NBUF - DIST), nchunk):
            wait_write(c % NBUF)

    return k


def kernel(input_ids, table, norm_weight):
    b, s = input_ids.shape
    n_rows = b * s
    idx3 = input_ids.astype(jnp.int32).reshape(NW, n_rows // (NW * CHUNK), CHUNK)
    out = lax.cond(
        jnp.all(norm_weight == 1.0),
        lambda: _make_kernel(n_rows, apply_w=False)(table, idx3, norm_weight),
        lambda: _make_kernel(n_rows, apply_w=True)(table, idx3, norm_weight),
    )
    return out.reshape(b, s, HIDDEN)
